# Initial kernel scaffold; baseline (speedup 1.0000x reference)
import jax
import jax.numpy as jnp
from jax.experimental import pallas as pl


def kernel(h, x, edge_attr, edge_index, W_e1, b_e1, W_e2, b_e2, W_c1, b_c1, W_c2, W_n1, b_n1, W_n2, b_n2, ln_g, ln_b):
    raise NotImplementedError("write your pallas kernel here")


# speedup vs baseline: 2.7532x; 2.7532x over previous
import functools

import jax
import jax.numpy as jnp
from jax import lax
from jax.experimental import pallas as pl
from jax.experimental.pallas import tpu as pltpu
from jax.experimental.pallas import tpu_sc as plsc

N = 10000
E = 320000
D = 128
ED = 16

NC = 2
NS = 16
NW = NC * NS
EPW = E // NW
CHUNK = 80
NCHUNK = EPW // CHUNK
NPT = N // NS

@functools.lru_cache(maxsize=None)
def _mesh():
    return plsc.VectorSubcoreMesh(core_axis_name="c", subcore_axis_name="s",
                                  num_cores=NC, num_subcores=NS)


def _pre_body(h_ref, w1a_ref, w1b_ref, a_ref, b_ref):
    h = h_ref[...]
    a_ref[...] = jnp.dot(h, w1a_ref[...], preferred_element_type=jnp.float32)
    b_ref[...] = jnp.dot(h, w1b_ref[...], preferred_element_type=jnp.float32)


def _pre(h, w1a, w1b):
    return pl.pallas_call(
        _pre_body,
        out_shape=[
            jax.ShapeDtypeStruct((N, D), jnp.float32),
            jax.ShapeDtypeStruct((N, D), jnp.float32),
        ],
    )(h, w1a, w1b)


@functools.lru_cache(maxsize=None)
def _build_gather():
  @functools.partial(
    pl.kernel,
    out_type=[
        jax.ShapeDtypeStruct((E, D), jnp.float32),
        jax.ShapeDtypeStruct((E, 8), jnp.float32),
        jax.ShapeDtypeStruct((E, 8), jnp.float32),
    ],
    mesh=_mesh(),
    scratch_types=[
        pltpu.VMEM((CHUNK,), jnp.int32),
        pltpu.VMEM((CHUNK,), jnp.int32),
        pltpu.VMEM((CHUNK, D), jnp.float32),
        pltpu.VMEM((CHUNK, D), jnp.float32),
        pltpu.VMEM((CHUNK, 8), jnp.float32),
        pltpu.VMEM((CHUNK, 8), jnp.float32),
        pltpu.SemaphoreType.DMA,
        pltpu.SemaphoreType.DMA,
        pltpu.SemaphoreType.DMA,
        pltpu.SemaphoreType.DMA,
    ],
    compiler_params=pltpu.CompilerParams(use_tc_tiling_on_sc=False),
  )
  def _gather(a_hbm, b_hbm, x8_hbm, row_hbm, col_hbm,
              s_out, xr_out, xc_out,
              idxr, idxc, bufa, bufb, bufxr, bufxc, sem0, sem1, sem2, sem3):
    wid = lax.axis_index("s") * NC + lax.axis_index("c")
    wbase = pl.multiple_of(wid * EPW, 8)

    def chunk_body(ci, carry):
        base = pl.multiple_of(wbase + ci * CHUNK, 8)
        pltpu.sync_copy(row_hbm.at[pl.ds(base, CHUNK)], idxr)
        pltpu.sync_copy(col_hbm.at[pl.ds(base, CHUNK)], idxc)
        cpa = pltpu.async_copy(a_hbm.at[idxr], bufa, sem0)
        cpb = pltpu.async_copy(b_hbm.at[idxc], bufb, sem1)
        cpxr = pltpu.async_copy(x8_hbm.at[idxr], bufxr, sem2)
        cpxc = pltpu.async_copy(x8_hbm.at[idxc], bufxc, sem3)
        cpa.wait()
        cpb.wait()

        def add_body(r, c2):
            for j in range(D // 16):
                sl = pl.ds(j * 16, 16)
                bufa[r, sl] = bufa[r, sl] + bufb[r, sl]
            return c2

        lax.fori_loop(0, CHUNK, add_body, 0, unroll=2)
        cpxr.wait()
        cpxc.wait()
        pltpu.sync_copy(bufa, s_out.at[pl.ds(base, CHUNK)])
        pltpu.sync_copy(bufxr, xr_out.at[pl.ds(base, CHUNK)])
        pltpu.sync_copy(bufxc, xc_out.at[pl.ds(base, CHUNK)])
        return carry

    lax.fori_loop(0, NCHUNK, chunk_body, 0)

  return _gather


BE = 2000


def _edge_body(s_ref, xr_ref, xc_ref, ea_ref, wr_ref, wa_ref, be1_ref,
               w2_ref, be2_ref, wc1_ref, bc1_ref, wc2_ref,
               m_ref, fv_ref):
    cd = xr_ref[...] - xc_ref[...]
    radial = jnp.sum(cd * cd, axis=1, keepdims=True)
    pre1 = (s_ref[...] + radial * wr_ref[...] + be1_ref[...]
            + jnp.dot(ea_ref[...], wa_ref[...],
                      preferred_element_type=jnp.float32))
    p = pre1 * jax.nn.sigmoid(pre1)
    q = jnp.dot(p, w2_ref[...], preferred_element_type=jnp.float32) + be2_ref[...]
    m = q * jax.nn.sigmoid(q)
    f = jnp.dot(m, wc1_ref[...], preferred_element_type=jnp.float32) + bc1_ref[...]
    f = f * jax.nn.sigmoid(f)
    fs = jnp.sum(f * wc2_ref[...], axis=1, keepdims=True)
    t = jnp.tanh(fs) * 0.1
    dist = jnp.sqrt(radial)
    m_ref[...] = m
    fv_ref[...] = cd * (t / (dist + 1e-8))


def _edge(s, xr, xc, ea, wr, wa, be1, w2, be2, wc1, bc1, wc2):
    grid = (E // BE,)
    blk = lambda i: (i, 0)
    zero = lambda i: (0, 0)
    return pl.pallas_call(
        _edge_body,
        grid=grid,
        in_specs=[
            pl.BlockSpec((BE, D), blk),
            pl.BlockSpec((BE, 8), blk),
            pl.BlockSpec((BE, 8), blk),
            pl.BlockSpec((BE, ED), blk),
            pl.BlockSpec((1, D), zero),
            pl.BlockSpec((ED, D), zero),
            pl.BlockSpec((1, D), zero),
            pl.BlockSpec((D, D), zero),
            pl.BlockSpec((1, D), zero),
            pl.BlockSpec((D, D), zero),
            pl.BlockSpec((1, D), zero),
            pl.BlockSpec((1, D), zero),
        ],
        out_specs=[
            pl.BlockSpec((BE, D), blk),
            pl.BlockSpec((BE, 8), blk),
        ],
        out_shape=[
            jax.ShapeDtypeStruct((E, D), jnp.float32),
            jax.ShapeDtypeStruct((E, 8), jnp.float32),
        ],
    )(s, xr, xc, ea, wr, wa, be1, w2, be2, wc1, bc1, wc2)


SCHUNK = 80
EPC = E // NC


@functools.lru_cache(maxsize=None)
def _build_scatter():
  @functools.partial(
    pl.kernel,
    out_type=[
        jax.ShapeDtypeStruct((NC, N, D), jnp.float32),
        jax.ShapeDtypeStruct((NC, N, 8), jnp.float32),
    ],
    mesh=_mesh(),
    scratch_types=[
        pltpu.VMEM((SCHUNK,), jnp.int32),
        pltpu.VMEM((SCHUNK, D), jnp.float32),
        pltpu.VMEM((SCHUNK, 8), jnp.float32),
        pltpu.VMEM_SHARED((N, D), jnp.float32),
        pltpu.VMEM_SHARED((N, 8), jnp.float32),
    ],
    compiler_params=pltpu.CompilerParams(use_tc_tiling_on_sc=False),
  )
  def _scatter(m_hbm, fv_hbm, row_hbm, zmi_hbm, zfv_hbm,
               mi_out, fv_out,
               idx, bufm, buff, MI, FV):
    c = lax.axis_index("c")
    s = lax.axis_index("s")
    rbase = s * NPT
    pltpu.sync_copy(zmi_hbm.at[pl.ds(rbase, NPT)], MI.at[pl.ds(rbase, NPT)])
    pltpu.sync_copy(zfv_hbm.at[pl.ds(rbase, NPT)], FV.at[pl.ds(rbase, NPT)])
    plsc.subcore_barrier()

    tbase = pl.multiple_of(c * EPC + s * EPW, 8)

    def body(ci, carry):
        eb = pl.multiple_of(tbase + ci * SCHUNK, 8)
        pltpu.sync_copy(row_hbm.at[pl.ds(eb, SCHUNK)], idx)
        pltpu.sync_copy(m_hbm.at[pl.ds(eb, SCHUNK)], bufm)
        pltpu.sync_copy(fv_hbm.at[pl.ds(eb, SCHUNK)], buff)
        pltpu.sync_copy(bufm, MI.at[idx], add=True)
        pltpu.sync_copy(buff, FV.at[idx], add=True)
        return carry

    lax.fori_loop(0, EPW // SCHUNK, body, 0)
    plsc.subcore_barrier()
    pltpu.sync_copy(MI.at[pl.ds(rbase, NPT)], mi_out.at[c, pl.ds(rbase, NPT)])
    pltpu.sync_copy(FV.at[pl.ds(rbase, NPT)], fv_out.at[c, pl.ds(rbase, NPT)])

  return _scatter


BN = 2000


def _node_body(h_ref, mi0_ref, mi1_ref, fv0_ref, fv1_ref, x8_ref,
               wn1h_ref, wn1m_ref, bn1_ref, wn2_ref, bn2_ref,
               lng_ref, lnb_ref,
               hnew_ref, xnew_ref):
    h = h_ref[...]
    mi = mi0_ref[...] + mi1_ref[...]
    u = (jnp.dot(h, wn1h_ref[...], preferred_element_type=jnp.float32)
         + jnp.dot(mi, wn1m_ref[...], preferred_element_type=jnp.float32)
         + bn1_ref[...])
    u = u * jax.nn.sigmoid(u)
    g = h + jnp.dot(u, wn2_ref[...], preferred_element_type=jnp.float32) + bn2_ref[...]
    mu = jnp.mean(g, axis=1, keepdims=True)
    gc = g - mu
    var = jnp.mean(gc * gc, axis=1, keepdims=True)
    hnew_ref[...] = gc / jnp.sqrt(var + 1e-5) * lng_ref[...] + lnb_ref[...]
    xnew_ref[...] = x8_ref[...] + fv0_ref[...] + fv1_ref[...]


def _node(h, mi0, mi1, fv0, fv1, x8, wn1h, wn1m, bn1, wn2, bn2, lng, lnb):
    grid = (N // BN,)
    blk = lambda i: (i, 0)
    zero = lambda i: (0, 0)
    return pl.pallas_call(
        _node_body,
        grid=grid,
        in_specs=[
            pl.BlockSpec((BN, D), blk),
            pl.BlockSpec((BN, D), blk),
            pl.BlockSpec((BN, D), blk),
            pl.BlockSpec((BN, 8), blk),
            pl.BlockSpec((BN, 8), blk),
            pl.BlockSpec((BN, 8), blk),
            pl.BlockSpec((D, D), zero),
            pl.BlockSpec((D, D), zero),
            pl.BlockSpec((1, D), zero),
            pl.BlockSpec((D, D), zero),
            pl.BlockSpec((1, D), zero),
            pl.BlockSpec((1, D), zero),
            pl.BlockSpec((1, D), zero),
        ],
        out_specs=[
            pl.BlockSpec((BN, D), blk),
            pl.BlockSpec((BN, 8), blk),
        ],
        out_shape=[
            jax.ShapeDtypeStruct((N, D), jnp.float32),
            jax.ShapeDtypeStruct((N, 8), jnp.float32),
        ],
    )(h, mi0, mi1, fv0, fv1, x8, wn1h, wn1m, bn1, wn2, bn2, lng, lnb)


def kernel(h, x, edge_attr, edge_index, W_e1, b_e1, W_e2, b_e2,
           W_c1, b_c1, W_c2, W_n1, b_n1, W_n2, b_n2, ln_g, ln_b):
    row = edge_index[0]
    col = edge_index[1]
    x8 = jnp.concatenate([x, jnp.zeros((N, 5), jnp.float32)], axis=1)

    A, B = _pre(h, W_e1[:D], W_e1[D:2 * D])
    S, XR, XC = _build_gather()(A, B, x8, row, col)

    wr = W_e1[2 * D:2 * D + 1]
    wa = W_e1[2 * D + 1:]
    M, FV = _edge(S, XR, XC, edge_attr,
                  wr, wa, b_e1.reshape(1, D),
                  W_e2, b_e2.reshape(1, D),
                  W_c1, b_c1.reshape(1, D),
                  W_c2.reshape(1, D))

    zmi = jnp.zeros((N, D), jnp.float32)
    zfv = jnp.zeros((N, 8), jnp.float32)
    MI, FVN = _build_scatter()(M, FV, row, zmi, zfv)

    h_new, x_new8 = _node(h, MI[0], MI[1], FVN[0], FVN[1], x8,
                          W_n1[:D], W_n1[D:], b_n1.reshape(1, D),
                          W_n2, b_n2.reshape(1, D),
                          ln_g.reshape(1, D), ln_b.reshape(1, D))
    return h_new, x_new8[:, :3]
